# Initial kernel scaffold; baseline (speedup 1.0000x reference)
import jax
import jax.numpy as jnp
from jax.experimental import pallas as pl


def kernel(x, table, gamma, beta):
    raise NotImplementedError("write your pallas kernel here")


# speedup vs baseline: 3.4725x; 3.4725x over previous
import functools

import jax
import jax.numpy as jnp
from jax import lax
from jax.experimental import pallas as pl
from jax.experimental.pallas import tpu as pltpu
from jax.experimental.pallas import tpu_sc as plsc

NUM_EMB = 64
EMB_DIM = 16
NC = 2
NS = 16
NW = NC * NS


def _ln_table_body(t_ref, g_ref, b_ref, o_ref):
    t = t_ref[...]
    m = jnp.mean(t, axis=1, keepdims=True)
    v = jnp.mean(jnp.square(t - m), axis=1, keepdims=True)
    o_ref[...] = (t - m) / jnp.sqrt(v + 1e-5) * g_ref[...] + b_ref[...]


def _ln_table(table, gamma, beta):
    return pl.pallas_call(
        _ln_table_body,
        out_shape=jax.ShapeDtypeStruct(table.shape, table.dtype),
    )(table, gamma, beta)


def _make_gather(B):
    assert B % (8 * NW) == 0
    bpw = B // NW
    C = 3200
    assert bpw % C == 0
    nchunk = bpw // C
    mesh = plsc.VectorSubcoreMesh(core_axis_name="c", subcore_axis_name="s")

    @functools.partial(
        pl.kernel,
        out_type=jax.ShapeDtypeStruct((B, EMB_DIM), jnp.float32),
        mesh=mesh,
        scratch_types=[
            pltpu.VMEM((C,), jnp.int32),
            pltpu.VMEM((C, EMB_DIM), jnp.float32),
            pltpu.SemaphoreType.DMA,
        ],
        compiler_params=pltpu.CompilerParams(use_tc_tiling_on_sc=False),
    )
    def gather(tab_hbm, idx_hbm, out_hbm, idx_v, rows_v, sem):
        wid = lax.axis_index("s") * NC + lax.axis_index("c")
        base = wid * bpw

        def chunk(i, carry):
            off = base + i * C
            pltpu.sync_copy(idx_hbm.at[pl.ds(off, C)], idx_v)
            pltpu.async_copy(tab_hbm.at[idx_v], rows_v, sem).wait()
            pltpu.sync_copy(rows_v, out_hbm.at[pl.ds(off, C)])
            return carry

        lax.fori_loop(0, nchunk, chunk, 0)

    return gather


def kernel(x, table, gamma, beta):
    Bx, L = x.shape
    normed = _ln_table(table, gamma.reshape(1, EMB_DIM), beta.reshape(1, EMB_DIM))
    flat = x.reshape(-1)
    out = _make_gather(flat.shape[0])(normed, flat)
    return out.reshape(Bx, L, EMB_DIM)


# speedup vs baseline: 3.4819x; 1.0027x over previous
import functools

import jax
import jax.numpy as jnp
from jax import lax
from jax.experimental import pallas as pl
from jax.experimental.pallas import tpu as pltpu
from jax.experimental.pallas import tpu_sc as plsc

NUM_EMB = 64
EMB_DIM = 16
NC = 2
NS = 16
NW = NC * NS


def _ln_table_body(t_ref, g_ref, b_ref, o_ref):
    t = t_ref[...]
    m = jnp.mean(t, axis=1, keepdims=True)
    v = jnp.mean(jnp.square(t - m), axis=1, keepdims=True)
    o_ref[...] = (t - m) / jnp.sqrt(v + 1e-5) * g_ref[...] + b_ref[...]


def _ln_table(table, gamma, beta):
    return pl.pallas_call(
        _ln_table_body,
        out_shape=jax.ShapeDtypeStruct(table.shape, table.dtype),
    )(table, gamma, beta)


def _make_gather(B):
    assert B % (8 * NW) == 0
    bpw = B // NW
    C = 800
    NB = 2
    assert bpw % (C * NB) == 0
    npairs = bpw // (C * NB)
    mesh = plsc.VectorSubcoreMesh(core_axis_name="c", subcore_axis_name="s")

    @functools.partial(
        pl.kernel,
        out_type=jax.ShapeDtypeStruct((B, EMB_DIM), jnp.float32),
        mesh=mesh,
        scratch_types=[
            pltpu.VMEM((bpw,), jnp.int32),
            pltpu.VMEM((NB, C, EMB_DIM), jnp.float32),
            pltpu.SemaphoreType.DMA,
            pltpu.SemaphoreType.DMA,
            pltpu.SemaphoreType.DMA,
            pltpu.SemaphoreType.DMA,
        ],
        compiler_params=pltpu.CompilerParams(use_tc_tiling_on_sc=False),
    )
    def gather(tab_hbm, idx_hbm, out_hbm, idx_v, rows_v, g0, g1, s0, s1):
        gsems = (g0, g1)
        ssems = (s0, s1)
        wid = lax.axis_index("s") * NC + lax.axis_index("c")
        base = wid * bpw
        pltpu.sync_copy(idx_hbm.at[pl.ds(base, bpw)], idx_v)

        def g_copy(i, b):
            return pltpu.make_async_copy(
                tab_hbm.at[idx_v.at[pl.ds(i * C, C)]], rows_v.at[b], gsems[b])

        def s_copy(i, b):
            return pltpu.make_async_copy(
                rows_v.at[b], out_hbm.at[pl.ds(base + i * C, C)], ssems[b])

        for b in range(NB):
            g_copy(b, b).start()

        def body(j, carry):
            for b in range(NB):
                i = j * NB + b
                g_copy(i, b).wait()
                s_copy(i, b).start()

            @pl.when(j < npairs - 1)
            def _():
                for b in range(NB):
                    i = j * NB + b
                    s_copy(i, b).wait()
                    g_copy(i + NB, b).start()

            return carry

        lax.fori_loop(0, npairs, body, 0)
        for b in range(NB):
            i = (npairs - 1) * NB + b
            s_copy(i, b).wait()

    return gather


def kernel(x, table, gamma, beta):
    Bx, L = x.shape
    normed = _ln_table(table, gamma.reshape(1, EMB_DIM), beta.reshape(1, EMB_DIM))
    flat = x.reshape(-1)
    out = _make_gather(flat.shape[0])(normed, flat)
    return out.reshape(Bx, L, EMB_DIM)


# speedup vs baseline: 6.8589x; 1.9699x over previous
import functools

import jax
import jax.numpy as jnp
from jax import lax
from jax.experimental import pallas as pl
from jax.experimental.pallas import tpu as pltpu
from jax.experimental.pallas import tpu_sc as plsc

NUM_EMB = 64
EMB_DIM = 16
NC = 2
NS = 16
NW = NC * NS


def _ln_table_body(t_ref, g_ref, b_ref, o_ref):
    t = t_ref[...]
    m = jnp.mean(t, axis=1, keepdims=True)
    v = jnp.mean(jnp.square(t - m), axis=1, keepdims=True)
    o_ref[...] = (t - m) / jnp.sqrt(v + 1e-5) * g_ref[...] + b_ref[...]


def _ln_table(table, gamma, beta):
    return pl.pallas_call(
        _ln_table_body,
        out_shape=jax.ShapeDtypeStruct(table.shape, table.dtype),
    )(table, gamma, beta)


def _make_gather(B):
    assert B % (8 * NW) == 0
    bpw = B // NW
    C = 800
    NB = 2
    assert bpw % (C * NB) == 0
    npairs = bpw // (C * NB)
    mesh = plsc.VectorSubcoreMesh(core_axis_name="c", subcore_axis_name="s")

    @functools.partial(
        pl.kernel,
        out_type=jax.ShapeDtypeStruct((B, EMB_DIM), jnp.float32),
        mesh=mesh,
        scratch_types=[
            pltpu.VMEM_SHARED((NUM_EMB, EMB_DIM), jnp.float32),
            pltpu.VMEM((bpw,), jnp.int32),
            pltpu.VMEM((NB, C, EMB_DIM), jnp.float32),
            pltpu.SemaphoreType.DMA,
            pltpu.SemaphoreType.DMA,
            pltpu.SemaphoreType.DMA,
            pltpu.SemaphoreType.DMA,
        ],
        compiler_params=pltpu.CompilerParams(use_tc_tiling_on_sc=False),
    )
    def gather(tab_hbm, idx_hbm, out_hbm, tab_v, idx_v, rows_v, g0, g1, s0, s1):
        gsems = (g0, g1)
        ssems = (s0, s1)
        wid = lax.axis_index("s") * NC + lax.axis_index("c")
        base = wid * bpw
        @pl.when(lax.axis_index("s") == 0)
        def _():
            pltpu.sync_copy(tab_hbm, tab_v)

        plsc.subcore_barrier()
        pltpu.sync_copy(idx_hbm.at[pl.ds(base, bpw)], idx_v)

        def g_copy(i, b):
            return pltpu.make_async_copy(
                tab_v.at[idx_v.at[pl.ds(i * C, C)]], rows_v.at[b], gsems[b])

        def s_copy(i, b):
            return pltpu.make_async_copy(
                rows_v.at[b], out_hbm.at[pl.ds(base + i * C, C)], ssems[b])

        for b in range(NB):
            g_copy(b, b).start()

        def body(j, carry):
            for b in range(NB):
                i = j * NB + b
                g_copy(i, b).wait()
                s_copy(i, b).start()

            @pl.when(j < npairs - 1)
            def _():
                for b in range(NB):
                    i = j * NB + b
                    s_copy(i, b).wait()
                    g_copy(i + NB, b).start()

            return carry

        lax.fori_loop(0, npairs, body, 0)
        for b in range(NB):
            i = (npairs - 1) * NB + b
            s_copy(i, b).wait()

    return gather


def kernel(x, table, gamma, beta):
    Bx, L = x.shape
    normed = _ln_table(table, gamma.reshape(1, EMB_DIM), beta.reshape(1, EMB_DIM))
    flat = x.reshape(-1)
    out = _make_gather(flat.shape[0])(normed, flat)
    return out.reshape(Bx, L, EMB_DIM)
